# baseline (speedup 1.0000x reference)
import functools

import jax
import jax.numpy as jnp
from jax import lax
from jax.experimental import pallas as pl
from jax.experimental.pallas import tpu as pltpu
from jax.experimental.pallas import tpu_sc as plsc

GAMMA_ = 0.01
B_, G_, S_, L_ = 64, 4096, 32, 4
LANES = 16
NSLAB = B_ // LANES
NPART = 32 // NSLAB
GPT = G_ // NPART
NGG = G_ // LANES
GGPT = GPT // LANES
GGC = 4
NCHUNK = GGPT // GGC
PITCH = LANES + 1

_LN2 = 0.6931471805599453
_INV_GAMMA = 1.0 / GAMMA_


def _sc_log(v):
    bits = lax.bitcast_convert_type(v, jnp.int32)
    e = ((bits >> 23) & 0xFF) - 127
    m = lax.bitcast_convert_type((bits & 0x007FFFFF) | 0x3F800000,
                                 jnp.float32)
    z = (m - 1.0) / (m + 1.0)
    z2 = z * z
    p = 1.0 / 9.0 + z2 * (1.0 / 11.0)
    p = 1.0 / 7.0 + z2 * p
    p = 1.0 / 5.0 + z2 * p
    p = 1.0 / 3.0 + z2 * p
    p = 1.0 + z2 * p
    return e.astype(jnp.float32) * _LN2 + 2.0 * z * p


def _clause_body(xt_hbm, idx_hbm, out_hbm, table_v, idx_v, conj_v, mx_v,
                 out_v, dma_sems):
    wid = lax.axis_index("s") * 2 + lax.axis_index("c")
    bslab = lax.rem(wid, NSLAB)
    gpart = lax.div(wid, NSLAB)
    g0 = gpart * GPT
    gg0 = gpart * GGPT
    b0 = bslab * LANES

    pltpu.sync_copy(xt_hbm.at[bslab], table_v)

    def issue_chunk(c, par):
        pltpu.async_copy(idx_hbm.at[pl.ds(gg0 + c * GGC, GGC)],
                         idx_v.at[par], dma_sems.at[par])

    def wait_chunk(c, par):
        pltpu.make_async_copy(idx_hbm.at[pl.ds(gg0 + c * GGC, GGC)],
                              idx_v.at[par], dma_sems.at[par]).wait()

    issue_chunk(0, 0)

    @pl.loop(0, NCHUNK)
    def _chunk(c):
        par = lax.rem(c, 2)

        @pl.when(c < NCHUNK - 1)
        def _():
            issue_chunk(c + 1, 1 - par)

        wait_chunk(c, par)

        @pl.loop(0, GGC)
        def _gg(ggl):
            init = tuple(jnp.full((LANES,), -1.0, jnp.float32)
                         for _ in range(LANES))

            cols = [jnp.full((LANES,), b, jnp.int32) for b in range(LANES)]

            @pl.loop(0, S_, init_carry=init)
            def mx_fin(s, mx):
                base = [idx_v[par, ggl, s, l, :] for l in range(L_)]
                new = []
                for b in range(LANES):
                    q0 = plsc.load_gather(table_v, [base[0], cols[b]])
                    q1 = plsc.load_gather(table_v, [base[1], cols[b]])
                    q2 = plsc.load_gather(table_v, [base[2], cols[b]])
                    q3 = plsc.load_gather(table_v, [base[3], cols[b]])
                    cv = (q0 * q1) * (q2 * q3)
                    conj_v[b, s, :] = cv
                    new.append(jnp.maximum(mx[b], cv))
                return tuple(new)

            for b in range(LANES):
                mx_v[b, :] = mx_fin[b]

            @pl.loop(0, LANES)
            def _batch(bi):
                mxv = mx_v[bi]
                acc = None
                for s in range(S_):
                    ev = jnp.exp((conj_v[bi, s] - mxv) * _INV_GAMMA)
                    acc = ev if acc is None else acc + ev
                res = mxv + GAMMA_ * _sc_log(acc)
                out_v[bi, pl.ds(ggl * LANES, LANES)] = res

        pltpu.sync_copy(out_v,
                        out_hbm.at[pl.ds(b0, LANES),
                                   pl.ds(g0 + c * GGC * LANES, GGC * LANES)])


@jax.jit
def kernel(x, I_i):
    xt = x.T.reshape(G_, NSLAB, LANES).transpose(1, 0, 2)
    xt = jnp.pad(xt, ((0, 0), (0, 0), (0, PITCH - LANES)))
    idx = (I_i.astype(jnp.int32)
           .reshape(NGG, LANES, S_, L_)
           .transpose(0, 2, 3, 1))

    mesh = plsc.VectorSubcoreMesh(core_axis_name="c", subcore_axis_name="s")
    run = functools.partial(
        pl.kernel,
        out_type=jax.ShapeDtypeStruct((B_, G_), jnp.float32),
        mesh=mesh,
        compiler_params=pltpu.CompilerParams(use_tc_tiling_on_sc=False,
                                             needs_layout_passes=False),
        scratch_types=[
            pltpu.VMEM((G_, PITCH), jnp.float32),
            pltpu.VMEM((2, GGC, S_, L_, LANES), jnp.int32),
            pltpu.VMEM((LANES, S_, LANES), jnp.float32),
            pltpu.VMEM((LANES, LANES), jnp.float32),
            pltpu.VMEM((LANES, GGC * LANES), jnp.float32),
            pltpu.SemaphoreType.DMA((2,)),
        ],
    )(_clause_body)
    return run(xt, idx)


# speedup vs baseline: 1.4373x; 1.4373x over previous
import functools

import jax
import jax.numpy as jnp
from jax import lax
from jax.experimental import pallas as pl
from jax.experimental.pallas import tpu as pltpu
from jax.experimental.pallas import tpu_sc as plsc

GAMMA_ = 0.01
B_, G_, S_, L_ = 64, 4096, 32, 4
LANES = 16
NSLAB = B_ // LANES
NPART = 32 // NSLAB
GPT = G_ // NPART
GC = 64
NCHUNK = GPT // GC
SL = S_ * L_

_LN2 = 0.6931471805599453
_INV_GAMMA = 1.0 / GAMMA_


def _sc_log(v):
    bits = lax.bitcast_convert_type(v, jnp.int32)
    e = ((bits >> 23) & 0xFF) - 127
    m = lax.bitcast_convert_type((bits & 0x007FFFFF) | 0x3F800000,
                                 jnp.float32)
    z = (m - 1.0) / (m + 1.0)
    z2 = z * z
    p = 1.0 / 9.0 + z2 * (1.0 / 11.0)
    p = 1.0 / 7.0 + z2 * p
    p = 1.0 / 5.0 + z2 * p
    p = 1.0 / 3.0 + z2 * p
    p = 1.0 + z2 * p
    return e.astype(jnp.float32) * _LN2 + 2.0 * z * p


def _clause_body(xt_hbm, idx_hbm, out_hbm, idx_v, gath_v, out_v,
                 idx_sems, gath_sems):
    wid = lax.axis_index("s") * 2 + lax.axis_index("c")
    bslab = lax.rem(wid, NSLAB)
    gpart = lax.div(wid, NSLAB)
    g0 = gpart * GPT
    b0 = bslab * LANES

    row_iota = lax.broadcasted_iota(jnp.int32, (LANES,), 0)

    def issue_idx(c, par):
        pltpu.async_copy(idx_hbm.at[bslab, pl.ds(g0 + c * GC, GC)],
                         idx_v.at[par], idx_sems.at[par])

    def wait_idx(c, par):
        pltpu.make_async_copy(idx_hbm.at[bslab, pl.ds(g0 + c * GC, GC)],
                              idx_v.at[par], idx_sems.at[par]).wait()

    def issue_gath(par, gl, gp):
        pltpu.async_copy(xt_hbm.at[idx_v.at[par, gl]],
                         gath_v.at[gp], gath_sems.at[gp])

    def wait_gath(par, gl, gp):
        pltpu.make_async_copy(xt_hbm.at[idx_v.at[par, gl]],
                              gath_v.at[gp], gath_sems.at[gp]).wait()

    issue_idx(0, 0)

    @pl.loop(0, NCHUNK)
    def _chunk(c):
        par = lax.rem(c, 2)

        @pl.when(c < NCHUNK - 1)
        def _():
            issue_idx(c + 1, 1 - par)

        wait_idx(c, par)
        issue_gath(par, 0, lax.rem(c * GC, 2))

        @pl.loop(0, GC)
        def _atom(gl):
            gp = lax.rem(c * GC + gl, 2)

            @pl.when(gl < GC - 1)
            def _():
                issue_gath(par, gl + 1, 1 - gp)

            wait_gath(par, gl, gp)

            conj = []
            mx = None
            for s in range(S_):
                c0 = gath_v[gp, 4 * s + 0]
                c1 = gath_v[gp, 4 * s + 1]
                c2 = gath_v[gp, 4 * s + 2]
                c3 = gath_v[gp, 4 * s + 3]
                cv = (c0 * c1) * (c2 * c3)
                conj.append(cv)
                mx = cv if mx is None else jnp.maximum(mx, cv)
            acc = None
            for s in range(S_):
                ev = jnp.exp((conj[s] - mx) * _INV_GAMMA)
                acc = ev if acc is None else acc + ev
            res = mx + GAMMA_ * _sc_log(acc)
            col = jnp.full((LANES,), gl, jnp.int32)
            plsc.store_scatter(out_v, [row_iota, col], res)

        pltpu.sync_copy(out_v,
                        out_hbm.at[pl.ds(b0, LANES),
                                   pl.ds(g0 + c * GC, GC)])


@jax.jit
def kernel(x, I_i):
    xt = (x.T.reshape(G_, NSLAB, LANES).transpose(1, 0, 2)
          .reshape(NSLAB * G_, LANES))
    idx = I_i.astype(jnp.int32).reshape(G_, SL)
    idx4 = idx[None] + (jnp.arange(NSLAB, dtype=jnp.int32)
                        * G_)[:, None, None]

    mesh = plsc.VectorSubcoreMesh(core_axis_name="c", subcore_axis_name="s")
    run = functools.partial(
        pl.kernel,
        out_type=jax.ShapeDtypeStruct((B_, G_), jnp.float32),
        mesh=mesh,
        compiler_params=pltpu.CompilerParams(use_tc_tiling_on_sc=False,
                                             needs_layout_passes=False),
        scratch_types=[
            pltpu.VMEM((2, GC, SL), jnp.int32),
            pltpu.VMEM((2, SL, LANES), jnp.float32),
            pltpu.VMEM((LANES, GC), jnp.float32),
            pltpu.SemaphoreType.DMA((2,)),
            pltpu.SemaphoreType.DMA((2,)),
        ],
    )(_clause_body)
    return run(xt, idx4)


# speedup vs baseline: 1.8772x; 1.3061x over previous
import functools

import jax
import jax.numpy as jnp
from jax import lax
from jax.experimental import pallas as pl
from jax.experimental.pallas import tpu as pltpu
from jax.experimental.pallas import tpu_sc as plsc

GAMMA_ = 0.01
B_, G_, S_, L_ = 64, 4096, 32, 4
LANES = 16
NSLAB = B_ // LANES
NPART = 32 // NSLAB
GPT = G_ // NPART
GC = 64
NCHUNK = GPT // GC
PW = S_ * L_ // 2

_LN2 = 0.6931471805599453
_INV_GAMMA = 1.0 / GAMMA_


def _sc_log(v):
    bits = lax.bitcast_convert_type(v, jnp.int32)
    e = ((bits >> 23) & 0xFF) - 127
    m = lax.bitcast_convert_type((bits & 0x007FFFFF) | 0x3F800000,
                                 jnp.float32)
    z = (m - 1.0) / (m + 1.0)
    z2 = z * z
    p = 1.0 / 9.0 + z2 * (1.0 / 11.0)
    p = 1.0 / 7.0 + z2 * p
    p = 1.0 / 5.0 + z2 * p
    p = 1.0 / 3.0 + z2 * p
    p = 1.0 + z2 * p
    return e.astype(jnp.float32) * _LN2 + 2.0 * z * p


def _clause_body(xt_hbm, idx_hbm, out_hbm, table_v, idx_v, conj_v, out_v,
                 idx_sems):
    wid = lax.axis_index("s") * 2 + lax.axis_index("c")
    bslab = lax.rem(wid, NSLAB)
    gpart = lax.div(wid, NSLAB)
    g0 = gpart * GPT
    b0 = bslab * LANES

    row_iota = lax.broadcasted_iota(jnp.int32, (LANES,), 0)

    pltpu.sync_copy(xt_hbm.at[pl.ds(bslab * (G_ * LANES), G_ * LANES)],
                    table_v)

    def issue_idx(c, par):
        pltpu.async_copy(idx_hbm.at[pl.ds(g0 + c * GC, GC)],
                         idx_v.at[par], idx_sems.at[par])

    def wait_idx(c, par):
        pltpu.make_async_copy(idx_hbm.at[pl.ds(g0 + c * GC, GC)],
                              idx_v.at[par], idx_sems.at[par]).wait()

    issue_idx(0, 0)

    @pl.loop(0, NCHUNK)
    def _chunk(c):
        par = lax.rem(c, 2)

        @pl.when(c < NCHUNK - 1)
        def _():
            issue_idx(c + 1, 1 - par)

        wait_idx(c, par)

        @pl.loop(0, GC)
        def _atom(gl):
            iv = [idx_v[par, gl, pl.ds(k * LANES, LANES)] for k in range(4)]

            conj = []
            mx = None
            for s in range(S_):
                e0 = iv[(2 * s) // LANES][(2 * s) % LANES]
                e1 = iv[(2 * s + 1) // LANES][(2 * s + 1) % LANES]
                c0 = table_v[pl.ds(e0 & 0xFFFF, LANES)]
                c1 = table_v[pl.ds(lax.shift_right_logical(e0, 16), LANES)]
                c2 = table_v[pl.ds(e1 & 0xFFFF, LANES)]
                c3 = table_v[pl.ds(lax.shift_right_logical(e1, 16), LANES)]
                cv = (c0 * c1) * (c2 * c3)
                conj_v[s, :] = cv
                mx = cv if mx is None else jnp.maximum(mx, cv)
            acc = None
            for s in range(S_):
                ev = jnp.exp((conj_v[s] - mx) * _INV_GAMMA)
                acc = ev if acc is None else acc + ev
            res = mx + GAMMA_ * _sc_log(acc)
            col = jnp.full((LANES,), gl, jnp.int32)
            plsc.store_scatter(out_v, [row_iota, col], res)

        pltpu.sync_copy(out_v,
                        out_hbm.at[pl.ds(b0, LANES),
                                   pl.ds(g0 + c * GC, GC)])


@jax.jit
def kernel(x, I_i):
    xt = (x.T.reshape(G_, NSLAB, LANES).transpose(1, 0, 2)
          .reshape(NSLAB * G_ * LANES))
    ii = I_i.astype(jnp.uint32).reshape(G_, PW, 2) * LANES
    idxp = (ii[:, :, 0] | (ii[:, :, 1] << 16)).astype(jnp.int32)

    mesh = plsc.VectorSubcoreMesh(core_axis_name="c", subcore_axis_name="s")
    run = functools.partial(
        pl.kernel,
        out_type=jax.ShapeDtypeStruct((B_, G_), jnp.float32),
        mesh=mesh,
        compiler_params=pltpu.CompilerParams(use_tc_tiling_on_sc=False,
                                             needs_layout_passes=False),
        scratch_types=[
            pltpu.VMEM((G_ * LANES,), jnp.float32),
            pltpu.VMEM((2, GC, PW), jnp.int32),
            pltpu.VMEM((S_, LANES), jnp.float32),
            pltpu.VMEM((LANES, GC), jnp.float32),
            pltpu.SemaphoreType.DMA((2,)),
        ],
    )(_clause_body)
    return run(xt, idxp)
